# Initial kernel scaffold; baseline (speedup 1.0000x reference)
import jax
import jax.numpy as jnp
from jax.experimental import pallas as pl


def kernel(x, W):
    raise NotImplementedError("write your pallas kernel here")


# speedup vs baseline: 10.7479x; 10.7479x over previous
import jax
import jax.numpy as jnp
from jax.experimental import pallas as pl

H = 14
W_ = 14
P = H * W_
WIN_HALF = 3
KK = 9
C = 384
OUT = 384
EPS = 1e-6
NEG = -1e30


def _dconv_body(x_ref, w_ref, o_ref):
    X = x_ref[0]
    nsq = jnp.sum(X * X, axis=0)
    n = jnp.sqrt(nsq)
    S = jax.lax.dot_general(X, X, (((0,), (0,)), ((), ())),
                            preferred_element_type=jnp.float32, precision=jax.lax.Precision.HIGHEST)
    denom = jnp.maximum(n[:, None] * n[None, :], EPS)
    sim = S / denom

    q_i = jax.lax.broadcasted_iota(jnp.int32, (P, P), 1)
    p_i = jax.lax.broadcasted_iota(jnp.int32, (P, P), 0)
    inwin = ((jnp.abs(p_i // W_ - q_i // W_) <= WIN_HALF)
             & (jnp.abs(p_i % W_ - q_i % W_) <= WIN_HALF))
    work = jnp.where(inwin, sim, NEG)

    selected = jnp.zeros((P, P), jnp.float32)
    for _ in range(KK):
        mx = jnp.max(work, axis=1, keepdims=True)
        cand = jnp.where(work == mx, q_i, P)
        amin = jnp.min(cand, axis=1, keepdims=True)
        hit = q_i == amin
        selected = selected + hit.astype(jnp.float32)
        work = jnp.where(hit, NEG, work)

    ltmat = (p_i <= q_i).astype(jnp.float32)
    rank = jax.lax.dot_general(selected, ltmat, (((1,), (0,)), ((), ())),
                               preferred_element_type=jnp.float32, precision=jax.lax.Precision.HIGHEST)

    acc = jnp.zeros((OUT, P), jnp.float32)
    for m in range(KK):
        pm = ((rank == float(m + 1)) & (selected > 0.5)).astype(jnp.float32)
        gm = jax.lax.dot_general(pm, X, (((1,), (1,)), ((), ())),
                                 preferred_element_type=jnp.float32, precision=jax.lax.Precision.HIGHEST)
        acc = acc + jax.lax.dot_general(
            w_ref[m], gm, (((0,), (1,)), ((), ())),
            preferred_element_type=jnp.float32, precision=jax.lax.Precision.HIGHEST)
    o_ref[0] = acc


def kernel(x, W):
    Bn = x.shape[0]
    x_flat = x.reshape(Bn, C, P)
    Wr = jnp.transpose(W.reshape(OUT, C, KK), (2, 1, 0))

    out = pl.pallas_call(
        _dconv_body,
        grid=(Bn,),
        in_specs=[
            pl.BlockSpec((1, C, P), lambda b: (b, 0, 0)),
            pl.BlockSpec((KK, C, OUT), lambda b: (0, 0, 0)),
        ],
        out_specs=pl.BlockSpec((1, OUT, P), lambda b: (b, 0, 0)),
        out_shape=jax.ShapeDtypeStruct((Bn, OUT, P), jnp.float32),
    )(x_flat, Wr)
    return out.reshape(Bn, OUT, H, W_)


# speedup vs baseline: 27.9556x; 2.6010x over previous
import jax
import jax.numpy as jnp
from jax.experimental import pallas as pl

H = 14
W_ = 14
P = H * W_
WIN_HALF = 3
KK = 9
C = 384
OUT = 384
EPS = 1e-6
NEG = -1e30


def _dconv_body(x_ref, w_ref, o_ref):
    X = x_ref[0]
    nsq = jnp.sum(X * X, axis=0)
    n = jnp.sqrt(nsq)
    S = jax.lax.dot_general(X, X, (((0,), (0,)), ((), ())),
                            preferred_element_type=jnp.float32, precision=jax.lax.Precision.HIGHEST)
    denom = jnp.maximum(n[:, None] * n[None, :], EPS)
    sim = S / denom

    q_i = jax.lax.broadcasted_iota(jnp.int32, (P, P), 1)
    p_i = jax.lax.broadcasted_iota(jnp.int32, (P, P), 0)
    inwin = ((jnp.abs(p_i // W_ - q_i // W_) <= WIN_HALF)
             & (jnp.abs(p_i % W_ - q_i % W_) <= WIN_HALF))
    work = jnp.where(inwin, sim, NEG)

    selected = jnp.zeros((P, P), jnp.float32)
    for _ in range(KK):
        mx = jnp.max(work, axis=1, keepdims=True)
        cand = jnp.where(work == mx, q_i, P)
        amin = jnp.min(cand, axis=1, keepdims=True)
        hit = q_i == amin
        selected = selected + hit.astype(jnp.float32)
        work = jnp.where(hit, NEG, work)

    ltmat = (p_i <= q_i).astype(jnp.float32)
    rank = jax.lax.dot_general(selected, ltmat, (((1,), (0,)), ((), ())),
                               preferred_element_type=jnp.float32)

    acc = jnp.zeros((OUT, P), jnp.float32)
    for m in range(KK):
        pm = ((rank == float(m + 1)) & (selected > 0.5)).astype(jnp.float32)
        gm = jax.lax.dot_general(pm, X, (((1,), (1,)), ((), ())),
                                 preferred_element_type=jnp.float32)
        acc = acc + jax.lax.dot_general(
            w_ref[m], gm, (((0,), (1,)), ((), ())),
            preferred_element_type=jnp.float32)
    o_ref[0] = acc


def kernel(x, W):
    Bn = x.shape[0]
    x_flat = x.reshape(Bn, C, P)
    Wr = jnp.transpose(W.reshape(OUT, C, KK), (2, 1, 0))

    out = pl.pallas_call(
        _dconv_body,
        grid=(Bn,),
        in_specs=[
            pl.BlockSpec((1, C, P), lambda b: (b, 0, 0)),
            pl.BlockSpec((KK, C, OUT), lambda b: (0, 0, 0)),
        ],
        out_specs=pl.BlockSpec((1, OUT, P), lambda b: (b, 0, 0)),
        out_shape=jax.ShapeDtypeStruct((Bn, OUT, P), jnp.float32),
    )(x_flat, Wr)
    return out.reshape(Bn, OUT, H, W_)


# speedup vs baseline: 29.7853x; 1.0654x over previous
import jax
import jax.numpy as jnp
from jax.experimental import pallas as pl

H = 14
W_ = 14
P = H * W_
WIN_HALF = 3
KK = 9
C = 384
OUT = 384
EPS = 1e-6
NEG = -1e30


def _dconv_one(X, w_ref):
    nsq = jnp.sum(X * X, axis=0)
    n = jnp.sqrt(nsq)
    S = jax.lax.dot_general(X, X, (((0,), (0,)), ((), ())),
                            preferred_element_type=jnp.float32, precision=jax.lax.Precision.HIGHEST)
    denom = jnp.maximum(n[:, None] * n[None, :], EPS)
    sim = S / denom

    q_i = jax.lax.broadcasted_iota(jnp.int32, (P, P), 1)
    p_i = jax.lax.broadcasted_iota(jnp.int32, (P, P), 0)
    inwin = ((jnp.abs(p_i // W_ - q_i // W_) <= WIN_HALF)
             & (jnp.abs(p_i % W_ - q_i % W_) <= WIN_HALF))
    work = jnp.where(inwin, sim, NEG)

    selected = jnp.zeros((P, P), jnp.float32)
    for _ in range(KK):
        mx = jnp.max(work, axis=1, keepdims=True)
        cand = jnp.where(work == mx, q_i, P)
        amin = jnp.min(cand, axis=1, keepdims=True)
        hit = q_i == amin
        selected = selected + hit.astype(jnp.float32)
        work = jnp.where(hit, NEG, work)

    ltmat = (p_i <= q_i).astype(jnp.float32)
    rank = jax.lax.dot_general(selected, ltmat, (((1,), (0,)), ((), ())),
                               preferred_element_type=jnp.float32)

    acc = jnp.zeros((OUT, P), jnp.float32)
    for m in range(KK):
        pm = ((rank == float(m + 1)) & (selected > 0.5)).astype(jnp.float32)
        gm = jax.lax.dot_general(pm, X, (((1,), (1,)), ((), ())),
                                 preferred_element_type=jnp.float32)
        acc = acc + jax.lax.dot_general(
            w_ref[m], gm, (((0,), (1,)), ((), ())),
            preferred_element_type=jnp.float32)
    return acc


BPG = 2


def _dconv_body(x_ref, w_ref, o_ref):
    for i in range(BPG):
        o_ref[i] = _dconv_one(x_ref[i], w_ref)


def kernel(x, W):
    Bn = x.shape[0]
    x_flat = x.reshape(Bn, C, P)
    Wr = jnp.transpose(W.reshape(OUT, C, KK), (2, 1, 0))

    out = pl.pallas_call(
        _dconv_body,
        grid=(Bn // BPG,),
        in_specs=[
            pl.BlockSpec((BPG, C, P), lambda b: (b, 0, 0)),
            pl.BlockSpec((KK, C, OUT), lambda b: (0, 0, 0)),
        ],
        out_specs=pl.BlockSpec((BPG, OUT, P), lambda b: (b, 0, 0)),
        out_shape=jax.ShapeDtypeStruct((Bn, OUT, P), jnp.float32),
    )(x_flat, Wr)
    return out.reshape(Bn, OUT, H, W_)


# speedup vs baseline: 36.3744x; 1.2212x over previous
import functools

import numpy as np
import jax
import jax.numpy as jnp
from jax import lax
from jax.experimental import pallas as pl
from jax.experimental.pallas import tpu as pltpu
from jax.experimental.pallas import tpu_sc as plsc

H = 14
W_ = 14
P = H * W_
WIN_HALF = 3
KK = 9
C = 384
OUT = 384
EPS = 1e-6
NEG = -1e30
SIMW = 256
NCAND = 64
BPG = 2


def _build_cidx():
    t = np.full((P, NCAND), -1, dtype=np.int32)
    for ki in range(H):
        for kj in range(W_):
            idx = [i * W_ + j
                   for i in range(H) for j in range(W_)
                   if abs(i - ki) <= WIN_HALF and abs(j - kj) <= WIN_HALF]
            t[ki * W_ + kj, :len(idx)] = np.array(idx, dtype=np.int32)
    return t


_CIDX = _build_cidx()


def _sims_body(x_ref, o_ref):
    for i in range(BPG):
        X = x_ref[i]
        n = jnp.sqrt(jnp.sum(X * X, axis=0))
        S = lax.dot_general(X, X, (((0,), (0,)), ((), ())),
                            preferred_element_type=jnp.float32,
                            precision=lax.Precision.HIGHEST)
        sim = S / jnp.maximum(n[:, None] * n[None, :], EPS)
        o_ref[i] = jnp.concatenate(
            [sim, jnp.full((P, SIMW - P), NEG, jnp.float32)], axis=1)


def _merge_desc(ak, av, bk, bv):
    brk = lax.rev(bk, (0,))
    brv = lax.rev(bv, (0,))
    m = ak >= brk
    hk = jnp.where(m, ak, brk)
    hv = jnp.where(m, av, brv)
    return plsc.sort_key_val(hk, hv, descending=True)


def _sc_topk_body(sims_hbm, cidx_hbm, out_hbm, rows_v, cidx_v, out_v):
    wid = lax.axis_index("s") * 2 + lax.axis_index("c")
    pltpu.sync_copy(sims_hbm.at[wid], rows_v)
    pltpu.sync_copy(cidx_hbm, cidx_v)
    lane = lax.broadcasted_iota(jnp.int32, (16,), 0)

    def body(p, carry):
        parts = []
        for k in range(4):
            ci = cidx_v[pl.ds(p * NCAND + 16 * k, 16)]
            valid = ci >= 0
            safe = jnp.where(valid, ci, 0)
            g = plsc.load_gather(rows_v, [p * SIMW + safe])
            key = jnp.where(valid, g, NEG)
            parts.append(plsc.sort_key_val(key, ci, descending=True))
        k01, v01 = _merge_desc(*parts[0], *parts[1])
        k23, v23 = _merge_desc(*parts[2], *parts[3])
        _, topv = _merge_desc(k01, v01, k23, v23)
        idx9 = jnp.where(lane < KK, topv, jnp.int32(2 ** 30))
        sidx, _ = plsc.sort_key_val(idx9, idx9)
        out_v[pl.ds(p * 16, 16)] = sidx
        return carry

    lax.fori_loop(0, P, body, 0)
    pltpu.sync_copy(out_v, out_hbm.at[wid])


def _sc_topk(sims, cidx):
    mesh = plsc.VectorSubcoreMesh(core_axis_name="c", subcore_axis_name="s")
    f = pl.kernel(
        _sc_topk_body,
        compiler_params=pltpu.CompilerParams(needs_layout_passes=False),
        out_type=jax.ShapeDtypeStruct((32, P * 16), jnp.int32),
        mesh=mesh,
        scratch_types=[
            pltpu.VMEM((P * SIMW,), jnp.float32),
            pltpu.VMEM((P * NCAND,), jnp.int32),
            pltpu.VMEM((P * 16,), jnp.int32),
        ],
    )
    return f(sims, cidx)


def _conv_body(x_ref, idx_ref, w_ref, o_ref):
    q_i = lax.broadcasted_iota(jnp.int32, (P, P), 1)
    for i in range(BPG):
        X = x_ref[i]
        acc = jnp.zeros((OUT, P), jnp.float32)
        for m in range(KK):
            im = idx_ref[i, :, m][:, None]
            pm = (q_i == im).astype(jnp.float32)
            gm = lax.dot_general(pm, X, (((1,), (1,)), ((), ())),
                                 preferred_element_type=jnp.float32)
            acc = acc + lax.dot_general(
                w_ref[m], gm, (((0,), (1,)), ((), ())),
                preferred_element_type=jnp.float32)
        o_ref[i] = acc


def kernel(x, W):
    Bn = x.shape[0]
    x_flat = x.reshape(Bn, C, P)
    Wr = jnp.transpose(W.reshape(OUT, C, KK), (2, 1, 0))
    cidx = jnp.asarray(_CIDX.reshape(-1))

    sims = pl.pallas_call(
        _sims_body,
        grid=(Bn // BPG,),
        in_specs=[pl.BlockSpec((BPG, C, P), lambda b: (b, 0, 0))],
        out_specs=pl.BlockSpec((BPG, P, SIMW), lambda b: (b, 0, 0)),
        out_shape=jax.ShapeDtypeStruct((Bn, P, SIMW), jnp.float32),
    )(x_flat)

    idx = _sc_topk(sims.reshape(Bn, P * SIMW), cidx)
    idx = idx.reshape(Bn, P, 16)

    out = pl.pallas_call(
        _conv_body,
        grid=(Bn // BPG,),
        in_specs=[
            pl.BlockSpec((BPG, C, P), lambda b: (b, 0, 0)),
            pl.BlockSpec((BPG, P, 16), lambda b: (b, 0, 0)),
            pl.BlockSpec((KK, C, OUT), lambda b: (0, 0, 0)),
        ],
        out_specs=pl.BlockSpec((BPG, OUT, P), lambda b: (b, 0, 0)),
        out_shape=jax.ShapeDtypeStruct((Bn, OUT, P), jnp.float32),
    )(x_flat, idx, Wr)
    return out.reshape(Bn, OUT, H, W_)
